# baseline (speedup 1.0000x reference)
import functools

import jax
import jax.numpy as jnp
from jax.experimental import pallas as pl
from jax.experimental.pallas import tpu as pltpu


def _softplus(x):
    ax = jnp.abs(x)
    return jnp.maximum(x, 0.0) + jnp.log1p(jnp.exp(-ax))


def _loss_body(nsteps, rows, k, out_ref, lab_ref, p_ref, loss_ref, acc_ref, keys_ref):
    b = pl.program_id(0)

    @pl.when(b == 0)
    def _init():
        acc_ref[...] = jnp.zeros_like(acc_ref)

    o = out_ref[...]
    perm = p_ref[...]

    def oplane(c):
        return o[:, c, :]

    def lplane(c):
        src = lab_ref[:, c, :, :]
        return jnp.transpose(src, (1, 0, 2)).reshape(rows, 96)

    x0 = oplane(0)
    first = lplane(0)
    posf = (first == 1.0).astype(jnp.float32)

    def col_sum(v):
        return jnp.sum(v, axis=0)

    acc_ref[0, :] += col_sum(posf)
    acc_ref[1, :] += col_sum(jnp.minimum(_softplus(-x0), 100.0) * posf)
    for c in range(1, 5):
        d = oplane(c) - lplane(c)
        a = jnp.abs(d)
        h = jnp.where(a < 1.0, 0.5 * d * d, a - 0.5)
        acc_ref[1 + c, :] += col_sum(h * posf)
    x5, x6, x7 = oplane(5), oplane(6), oplane(7)
    m = jnp.maximum(jnp.maximum(x5, x6), x7)
    lse = m + jnp.log(jnp.exp(x5 - m) + jnp.exp(x6 - m) + jnp.exp(x7 - m))
    ml = lplane(5)
    picked = jnp.where(ml == 0.0, x5, jnp.where(ml == 1.0, x6, x7)) - lse
    acc_ref[6, :] += col_sum(picked * posf)

    s = jnp.where(first == 0.0, x0, -jnp.inf)
    bb = jax.lax.bitcast_convert_type(s, jnp.int32)
    key = jnp.where(bb < 0, bb ^ jnp.int32(0x7FFFFFFF), bb)
    keys_ref[pl.ds(b * rows, rows), :] = key

    @pl.when(b == nsteps - 1)
    def _finalize():
        def bs_body(_, carry):
            lo, hi = carry
            span = lo ^ hi
            mid = (lo & hi) + (span >> 1) + (span & 1)
            cnt = jnp.sum((keys_ref[...] >= mid).astype(jnp.int32))
            ge = cnt >= k
            return jnp.where(ge, mid, lo), jnp.where(ge, hi, mid - 1)

        lo0 = jnp.int32(-(2**31))
        hi0 = jnp.int32(2**31 - 1)
        tau, _ = jax.lax.fori_loop(0, 32, bs_body, (lo0, hi0))

        keys = keys_ref[...]
        n_gt = jnp.sum((keys > tau).astype(jnp.int32))
        r = (k - n_gt).astype(jnp.float32)
        vbits = jnp.where(keys < 0, keys ^ jnp.int32(0x7FFFFFFF), keys)
        v = jax.lax.bitcast_convert_type(vbits, jnp.float32)
        fv = jnp.where(v == -jnp.inf, 100.0, jnp.minimum(_softplus(v), 100.0))
        s_gt = jnp.sum(jnp.where(keys > tau, fv, 0.0))
        f_tau = jnp.max(jnp.where(keys == tau, fv, -1.0))
        d_sum = s_gt + r * f_tau

        pc = jnp.sum(acc_ref[0, :])
        a_sum = jnp.sum(acc_ref[1, :])
        b_sum = (
            jnp.sum(acc_ref[2, :])
            + jnp.sum(acc_ref[3, :])
            + jnp.sum(acc_ref[4, :])
            + jnp.sum(acc_ref[5, :])
        )
        c_sum = jnp.sum(acc_ref[6, :])
        loss_ref[0, 0] = (
            0.5 * a_sum / pc + 0.5 * d_sum / jnp.float32(k) + (b_sum - c_sum) / pc
        )


def kernel(output, labels):
    nb, nh, nw, na, c_out = output.shape
    c_lab = labels.shape[4]
    n = nb * nh * nw * na
    k = min(32 * nb, n)
    rows = nh * na

    out_p = output.transpose(0, 1, 3, 4, 2).reshape(nb * rows, c_out, nw)
    lab_p = labels.transpose(0, 3, 4, 1, 2).reshape(nb * na, c_lab, nh, nw)

    i_idx = jax.lax.broadcasted_iota(jnp.int32, (rows, rows), 0)
    j_idx = jax.lax.broadcasted_iota(jnp.int32, (rows, rows), 1)
    perm = (j_idx == (i_idx % na) * nh + i_idx // na).astype(jnp.float32)

    body = functools.partial(_loss_body, nb, rows, k)
    loss = pl.pallas_call(
        body,
        grid=(nb,),
        in_specs=[
            pl.BlockSpec((rows, c_out, nw), lambda i: (i, 0, 0)),
            pl.BlockSpec((na, c_lab, nh, nw), lambda i: (i, 0, 0, 0)),
            pl.BlockSpec((rows, rows), lambda i: (0, 0)),
        ],
        out_specs=pl.BlockSpec(memory_space=pltpu.SMEM),
        out_shape=jax.ShapeDtypeStruct((1, 1), jnp.float32),
        scratch_shapes=[
            pltpu.VMEM((8, nw), jnp.float32),
            pltpu.VMEM((nb * rows, nw), jnp.int32),
        ],
        compiler_params=pltpu.CompilerParams(
            dimension_semantics=("arbitrary",),
        ),
    )(out_p, lab_p, perm)
    return loss.reshape(())


# speedup vs baseline: 1.9345x; 1.9345x over previous
import functools

import jax
import jax.numpy as jnp
from jax.experimental import pallas as pl
from jax.experimental.pallas import tpu as pltpu


def _softplus(x):
    ax = jnp.abs(x)
    return jnp.maximum(x, 0.0) + jnp.log1p(jnp.exp(-ax))


def _loss_body(nsteps, rows, k, out_ref, lab_ref, p_ref, loss_ref, acc_ref, keys_ref):
    b = pl.program_id(0)

    @pl.when(b == 0)
    def _init():
        acc_ref[...] = jnp.zeros_like(acc_ref)

    o = out_ref[...]
    perm = p_ref[...]

    def oplane(c):
        return o[:, c, :]

    def lplane(c):
        src = lab_ref[:, c, :, :].reshape(rows, 96)
        return jnp.dot(perm, src, preferred_element_type=jnp.float32)

    x0 = oplane(0)
    first = lplane(0)
    posf = (first == 1.0).astype(jnp.float32)

    def col_sum(v):
        return jnp.sum(v, axis=0)

    acc_ref[0, :] += col_sum(posf)
    acc_ref[1, :] += col_sum(jnp.minimum(_softplus(-x0), 100.0) * posf)
    for c in range(1, 5):
        d = oplane(c) - lplane(c)
        a = jnp.abs(d)
        h = jnp.where(a < 1.0, 0.5 * d * d, a - 0.5)
        acc_ref[1 + c, :] += col_sum(h * posf)
    x5, x6, x7 = oplane(5), oplane(6), oplane(7)
    m = jnp.maximum(jnp.maximum(x5, x6), x7)
    lse = m + jnp.log(jnp.exp(x5 - m) + jnp.exp(x6 - m) + jnp.exp(x7 - m))
    ml = lplane(5)
    picked = jnp.where(ml == 0.0, x5, jnp.where(ml == 1.0, x6, x7)) - lse
    acc_ref[6, :] += col_sum(picked * posf)

    s = jnp.where(first == 0.0, x0, -jnp.inf)
    bb = jax.lax.bitcast_convert_type(s, jnp.int32)
    key = jnp.where(bb < 0, bb ^ jnp.int32(0x7FFFFFFF), bb)
    keys_ref[pl.ds(b * rows, rows), :] = key

    @pl.when(b == nsteps - 1)
    def _finalize():
        def bs_body(_, carry):
            lo, hi = carry
            span = lo ^ hi
            mid = (lo & hi) + (span >> 1) + (span & 1)
            cnt = jnp.sum((keys_ref[...] >= mid).astype(jnp.int32))
            ge = cnt >= k
            return jnp.where(ge, mid, lo), jnp.where(ge, hi, mid - 1)

        lo0 = jnp.int32(-(2**31))
        hi0 = jnp.int32(2**31 - 1)
        tau, _ = jax.lax.fori_loop(0, 32, bs_body, (lo0, hi0))

        keys = keys_ref[...]
        n_gt = jnp.sum((keys > tau).astype(jnp.int32))
        r = (k - n_gt).astype(jnp.float32)
        vbits = jnp.where(keys < 0, keys ^ jnp.int32(0x7FFFFFFF), keys)
        v = jax.lax.bitcast_convert_type(vbits, jnp.float32)
        fv = jnp.where(v == -jnp.inf, 100.0, jnp.minimum(_softplus(v), 100.0))
        s_gt = jnp.sum(jnp.where(keys > tau, fv, 0.0))
        f_tau = jnp.max(jnp.where(keys == tau, fv, -1.0))
        d_sum = s_gt + r * f_tau

        pc = jnp.sum(acc_ref[0, :])
        a_sum = jnp.sum(acc_ref[1, :])
        b_sum = (
            jnp.sum(acc_ref[2, :])
            + jnp.sum(acc_ref[3, :])
            + jnp.sum(acc_ref[4, :])
            + jnp.sum(acc_ref[5, :])
        )
        c_sum = jnp.sum(acc_ref[6, :])
        loss_ref[0, 0] = (
            0.5 * a_sum / pc + 0.5 * d_sum / jnp.float32(k) + (b_sum - c_sum) / pc
        )


def kernel(output, labels):
    nb, nh, nw, na, c_out = output.shape
    c_lab = labels.shape[4]
    n = nb * nh * nw * na
    k = min(32 * nb, n)
    rows = nh * na

    out_p = output.transpose(0, 1, 3, 4, 2).reshape(nb * rows, c_out, nw)
    lab_p = labels.transpose(0, 3, 4, 1, 2).reshape(nb * na, c_lab, nh, nw)

    i_idx = jax.lax.broadcasted_iota(jnp.int32, (rows, rows), 0)
    j_idx = jax.lax.broadcasted_iota(jnp.int32, (rows, rows), 1)
    perm = (j_idx == (i_idx % na) * nh + i_idx // na).astype(jnp.float32)

    body = functools.partial(_loss_body, nb, rows, k)
    loss = pl.pallas_call(
        body,
        grid=(nb,),
        in_specs=[
            pl.BlockSpec((rows, c_out, nw), lambda i: (i, 0, 0)),
            pl.BlockSpec((na, c_lab, nh, nw), lambda i: (i, 0, 0, 0)),
            pl.BlockSpec((rows, rows), lambda i: (0, 0)),
        ],
        out_specs=pl.BlockSpec(memory_space=pltpu.SMEM),
        out_shape=jax.ShapeDtypeStruct((1, 1), jnp.float32),
        scratch_shapes=[
            pltpu.VMEM((8, nw), jnp.float32),
            pltpu.VMEM((nb * rows, nw), jnp.int32),
        ],
        compiler_params=pltpu.CompilerParams(
            dimension_semantics=("arbitrary",),
        ),
    )(out_p, lab_p, perm)
    return loss.reshape(())
